# baseline (speedup 1.0000x reference)
import numpy as np
import jax
import jax.numpy as jnp
from jax.experimental import pallas as pl
from jax.experimental.pallas import tpu as pltpu

_WS = 11
_HALF = _WS // 2
_SIGMA = 1.5
_C1 = 0.01 ** 2
_C2 = 0.03 ** 2


def _gauss_taps():
    x = np.arange(_WS, dtype=np.float64) - _WS // 2
    g = np.exp(-(x * x) / (2.0 * _SIGMA * _SIGMA))
    return (g / g.sum()).astype(np.float32)


def _lane_blur_mat(g, n):
    ws = g.shape[0]
    half = ws // 2
    i = np.arange(n)[:, None]
    o = np.arange(n)[None, :]
    j = i - o + half
    band = np.where((j >= 0) & (j < ws), g[np.clip(j, 0, ws - 1)], 0.0)
    return band.astype(np.float32)


def _make_body(taps, D, H, W):
    w_consts = [float(v) for v in taps]

    def _swap(a):
        return jnp.concatenate([a[:, W:], a[:, :W]], axis=1)

    def body(xy_ref, tw2_ref, ah_ref, map_ref, sum_ref, buf_ref):

        def blur(s):
            p = xy_ref[s]
            q = jnp.concatenate([p, p * p, p * _swap(p)], axis=0)
            t = jnp.dot(q, tw2_ref[...],
                        preferred_element_type=jnp.float32)
            ah = ah_ref[...]
            for gi in range(3):
                buf_ref[s, gi * H:(gi + 1) * H] = jnp.dot(
                    ah, t[gi * H:(gi + 1) * H],
                    preferred_element_type=jnp.float32)

        def tap_acc(dd, g):
            acc = None
            for k in range(_HALF):
                jlo, jhi = dd - _HALF + k, dd + _HALF - k
                lo_ok, hi_ok = jlo >= 0, jhi < D
                if lo_ok and hi_ok:
                    t = w_consts[k] * (buf_ref[jlo, g * H:(g + 1) * H]
                                       + buf_ref[jhi, g * H:(g + 1) * H])
                elif lo_ok:
                    t = w_consts[k] * buf_ref[jlo, g * H:(g + 1) * H]
                elif hi_ok:
                    t = w_consts[k] * buf_ref[jhi, g * H:(g + 1) * H]
                else:
                    continue
                acc = t if acc is None else acc + t
            t = w_consts[_HALF] * buf_ref[dd, g * H:(g + 1) * H]
            return t if acc is None else acc + t

        for s in range(_HALF + 2):
            blur(s)

        vsum = jnp.zeros((H, W), jnp.float32)
        for d in range(0, D, 2):
            for i in range(2):
                p0 = tap_acc(d + i, 0)
                p1 = tap_acc(d + i, 1)
                p2 = tap_acc(d + i, 2)

                prod = p0 * _swap(p0)
                sq = p0 * p0
                sqs = sq + _swap(sq)
                dif = p1 - sq
                sig = dif + _swap(dif)
                num = (2.0 * prod + _C1) * (2.0 * (p2 - prod) + _C2)
                den = (sqs + _C1) * (sig + _C2)
                smap = num * pl.reciprocal(den, approx=True)

                sm = smap[:, :W]
                map_ref[d + i] = -sm
                vsum = vsum + sm

            for s_next in (d + _HALF + 2, d + _HALF + 3):
                if s_next < D:
                    blur(s_next)

        sum_ref[0, 0] = jnp.sum(vsum)

    return body


def _ssim3d(img1, img2):
    N, C, D, H, W = img1.shape
    B = N * C
    g = _gauss_taps()
    tw = _lane_blur_mat(g, W)
    tw2 = np.zeros((2 * W, 2 * W), np.float32)
    tw2[:W, :W] = tw
    tw2[W:, W:] = tw
    ah = np.ascontiguousarray(_lane_blur_mat(g, H).T)

    x = img1.astype(jnp.float32).reshape(B, D, H, W)
    y = img2.astype(jnp.float32).reshape(B, D, H, W)
    xy = jnp.concatenate([x, y], axis=-1)

    body = _make_body(tuple(float(v) for v in g), D, H, W)

    def run(xy_l, tw2_l, ah_l):
        b_l = xy_l.shape[0]
        return pl.pallas_call(
            body,
            grid=(b_l,),
            in_specs=[
                pl.BlockSpec((None, D, H, 2 * W), lambda b: (b, 0, 0, 0)),
                pl.BlockSpec((2 * W, 2 * W), lambda b: (0, 0)),
                pl.BlockSpec((H, H), lambda b: (0, 0)),
            ],
            out_specs=(
                pl.BlockSpec((None, D, H, W), lambda b: (b, 0, 0, 0)),
                pl.BlockSpec((None, 1, 1), lambda b: (b, 0, 0),
                             memory_space=pltpu.MemorySpace.SMEM),
            ),
            out_shape=(
                jax.ShapeDtypeStruct((b_l, D, H, W), jnp.float32),
                jax.ShapeDtypeStruct((b_l, 1, 1), jnp.float32),
            ),
            scratch_shapes=[
                pltpu.VMEM((D, 3 * H, 2 * W), jnp.float32),
            ],
            compiler_params=pltpu.CompilerParams(
                dimension_semantics=("parallel",),
                vmem_limit_bytes=56 * 1024 * 1024,
            ),
        )(xy_l, tw2_l, ah_l)

    devs = jax.devices()
    n_dev = 2 if (len(devs) >= 2 and B % 2 == 0) else 1
    if n_dev == 2:
        from jax.sharding import Mesh, PartitionSpec as P
        try:
            from jax.experimental.shard_map import shard_map
        except ImportError:
            shard_map = jax.shard_map
        mesh = Mesh(np.asarray(devs[:2]), ("b",))
        neg_map, psums = shard_map(
            run, mesh=mesh,
            in_specs=(P("b"), P(), P()),
            out_specs=(P("b"), P("b")),
            check_rep=False,
        )(jnp.asarray(xy), jnp.asarray(tw2), jnp.asarray(ah))
    else:
        neg_map, psums = run(jnp.asarray(xy), jnp.asarray(tw2), jnp.asarray(ah))

    mean = jnp.sum(psums) / float(B * D * H * W)
    return 1.0 - mean, neg_map.reshape(N, C, D, H, W)


def kernel(img1, img2):
    return _ssim3d(img1, img2)


# speedup vs baseline: 4.6706x; 4.6706x over previous
import numpy as np
import jax
import jax.numpy as jnp
from jax.experimental import pallas as pl
from jax.experimental.pallas import tpu as pltpu

_WS = 11
_HALF = _WS // 2
_SIGMA = 1.5
_C1 = 0.01 ** 2
_C2 = 0.03 ** 2


def _gauss_taps():
    x = np.arange(_WS, dtype=np.float64) - _WS // 2
    g = np.exp(-(x * x) / (2.0 * _SIGMA * _SIGMA))
    return (g / g.sum()).astype(np.float32)


def _lane_blur_mat(g, n):
    ws = g.shape[0]
    half = ws // 2
    i = np.arange(n)[:, None]
    o = np.arange(n)[None, :]
    j = i - o + half
    band = np.where((j >= 0) & (j < ws), g[np.clip(j, 0, ws - 1)], 0.0)
    return band.astype(np.float32)


def _make_body(taps, D, H, W):
    w_consts = [float(v) for v in taps]

    def _swap(a):
        return jnp.concatenate([a[:, W:], a[:, :W]], axis=1)

    def body(xy_ref, tw2_ref, ah_ref, map_ref, sum_ref, buf_ref):

        def blur(s):
            p = xy_ref[s]
            q = jnp.concatenate([p, p * p, p * _swap(p)], axis=0)
            t = jnp.dot(q, tw2_ref[...],
                        preferred_element_type=jnp.float32)
            ah = ah_ref[...]
            for gi in range(3):
                buf_ref[s, gi * H:(gi + 1) * H] = jnp.dot(
                    ah, t[gi * H:(gi + 1) * H],
                    preferred_element_type=jnp.float32)

        def tap_acc(dd, g):
            acc = None
            for k in range(_HALF):
                jlo, jhi = dd - _HALF + k, dd + _HALF - k
                lo_ok, hi_ok = jlo >= 0, jhi < D
                if lo_ok and hi_ok:
                    t = w_consts[k] * (buf_ref[jlo, g * H:(g + 1) * H]
                                       + buf_ref[jhi, g * H:(g + 1) * H])
                elif lo_ok:
                    t = w_consts[k] * buf_ref[jlo, g * H:(g + 1) * H]
                elif hi_ok:
                    t = w_consts[k] * buf_ref[jhi, g * H:(g + 1) * H]
                else:
                    continue
                acc = t if acc is None else acc + t
            t = w_consts[_HALF] * buf_ref[dd, g * H:(g + 1) * H]
            return t if acc is None else acc + t

        for s in range(_HALF + 2):
            blur(s)

        vsum = jnp.zeros((H, W), jnp.float32)
        for d in range(0, D, 2):
            for i in range(2):
                p0 = tap_acc(d + i, 0)
                p1 = tap_acc(d + i, 1)
                p2 = tap_acc(d + i, 2)

                prod = p0 * _swap(p0)
                sq = p0 * p0
                sqs = sq + _swap(sq)
                dif = p1 - sq
                sig = dif + _swap(dif)
                num = (2.0 * prod + _C1) * (2.0 * (p2 - prod) + _C2)
                den = (sqs + _C1) * (sig + _C2)
                smap = num * pl.reciprocal(den, approx=True)

                sm = smap[:, :W]
                map_ref[d + i] = -sm
                vsum = vsum + sm

            for s_next in (d + _HALF + 2, d + _HALF + 3):
                if s_next < D:
                    blur(s_next)

        sum_ref[0, 0] = jnp.sum(vsum)

    return body


def _ssim3d(img1, img2):
    N, C, D, H, W = img1.shape
    B = N * C
    g = _gauss_taps()
    tw = _lane_blur_mat(g, W)
    tw2 = np.zeros((2 * W, 2 * W), np.float32)
    tw2[:W, :W] = tw
    tw2[W:, W:] = tw
    ah = np.ascontiguousarray(_lane_blur_mat(g, H).T)

    x = img1.astype(jnp.float32).reshape(B, D, H, W)
    y = img2.astype(jnp.float32).reshape(B, D, H, W)
    xy = jnp.concatenate([x, y], axis=-1)

    body = _make_body(tuple(float(v) for v in g), D, H, W)

    neg_map, psums = pl.pallas_call(
        body,
        grid=(B,),
        in_specs=[
            pl.BlockSpec((None, D, H, 2 * W), lambda b: (b, 0, 0, 0)),
            pl.BlockSpec((2 * W, 2 * W), lambda b: (0, 0)),
            pl.BlockSpec((H, H), lambda b: (0, 0)),
        ],
        out_specs=(
            pl.BlockSpec((None, D, H, W), lambda b: (b, 0, 0, 0)),
            pl.BlockSpec((None, 1, 1), lambda b: (b, 0, 0),
                         memory_space=pltpu.MemorySpace.SMEM),
        ),
        out_shape=(
            jax.ShapeDtypeStruct((B, D, H, W), jnp.float32),
            jax.ShapeDtypeStruct((B, 1, 1), jnp.float32),
        ),
        scratch_shapes=[
            pltpu.VMEM((D, 3 * H, 2 * W), jnp.float32),
        ],
        compiler_params=pltpu.CompilerParams(
            dimension_semantics=("parallel",),
            vmem_limit_bytes=56 * 1024 * 1024,
        ),
    )(jnp.asarray(xy), jnp.asarray(tw2), jnp.asarray(ah))

    mean = jnp.sum(psums) / float(B * D * H * W)
    return 1.0 - mean, neg_map.reshape(N, C, D, H, W)


def kernel(img1, img2):
    return _ssim3d(img1, img2)


# speedup vs baseline: 4.7541x; 1.0179x over previous
import numpy as np
import jax
import jax.numpy as jnp
from jax.experimental import pallas as pl
from jax.experimental.pallas import tpu as pltpu

_WS = 11
_HALF = _WS // 2
_SIGMA = 1.5
_C1 = 0.01 ** 2
_C2 = 0.03 ** 2


def _gauss_taps():
    x = np.arange(_WS, dtype=np.float64) - _WS // 2
    g = np.exp(-(x * x) / (2.0 * _SIGMA * _SIGMA))
    return (g / g.sum()).astype(np.float32)


def _lane_blur_mat(g, n):
    ws = g.shape[0]
    half = ws // 2
    i = np.arange(n)[:, None]
    o = np.arange(n)[None, :]
    j = i - o + half
    band = np.where((j >= 0) & (j < ws), g[np.clip(j, 0, ws - 1)], 0.0)
    return band.astype(np.float32)


def _make_body(taps, D, H, W):
    w_consts = [float(v) for v in taps]

    def _swap(a):
        return jnp.concatenate([a[:, W:], a[:, :W]], axis=1)

    def body(xy_ref, tw2_ref, ah_ref, map_ref, sum_ref, buf_ref):

        def blur(s):
            p = xy_ref[s]
            q = jnp.concatenate([p, p * p, p * _swap(p)], axis=0)
            t = jnp.dot(q, tw2_ref[...],
                        preferred_element_type=jnp.float32)
            ah = ah_ref[...]
            for gi in range(3):
                buf_ref[s, gi * H:(gi + 1) * H] = jnp.dot(
                    ah, t[gi * H:(gi + 1) * H],
                    preferred_element_type=jnp.float32)

        def tap_acc(dd, g):
            acc = None
            for k in range(_HALF):
                jlo, jhi = dd - _HALF + k, dd + _HALF - k
                lo_ok, hi_ok = jlo >= 0, jhi < D
                if lo_ok and hi_ok:
                    t = w_consts[k] * (buf_ref[jlo, g * H:(g + 1) * H]
                                       + buf_ref[jhi, g * H:(g + 1) * H])
                elif lo_ok:
                    t = w_consts[k] * buf_ref[jlo, g * H:(g + 1) * H]
                elif hi_ok:
                    t = w_consts[k] * buf_ref[jhi, g * H:(g + 1) * H]
                else:
                    continue
                acc = t if acc is None else acc + t
            t = w_consts[_HALF] * buf_ref[dd, g * H:(g + 1) * H]
            return t if acc is None else acc + t

        for s in range(_HALF + 2):
            blur(s)

        vsum = jnp.zeros((H, 2 * W), jnp.float32)
        for d in range(0, D, 2):
            a0 = tap_acc(d, 0)
            b0 = tap_acc(d + 1, 0)
            a1 = tap_acc(d, 1)
            b1 = tap_acc(d + 1, 1)
            a2 = tap_acc(d, 2)
            b2 = tap_acc(d + 1, 2)

            mu1 = jnp.concatenate([a0[:, :W], b0[:, :W]], axis=1)
            mu2 = jnp.concatenate([a0[:, W:], b0[:, W:]], axis=1)
            e1 = jnp.concatenate([a1[:, :W], b1[:, :W]], axis=1)
            e2 = jnp.concatenate([a1[:, W:], b1[:, W:]], axis=1)
            exy = jnp.concatenate([a2[:, :W], b2[:, :W]], axis=1)

            mm = mu1 * mu2
            sq1 = mu1 * mu1
            sq2 = mu2 * mu2
            sqs = sq1 + sq2
            sig = (e1 - sq1) + (e2 - sq2)
            num = (2.0 * mm + _C1) * (2.0 * (exy - mm) + _C2)
            den = (sqs + _C1) * (sig + _C2)
            smap = num * pl.reciprocal(den, approx=True)

            map_ref[d] = -smap[:, :W]
            map_ref[d + 1] = -smap[:, W:]
            vsum = vsum + smap

            for s_next in (d + _HALF + 2, d + _HALF + 3):
                if s_next < D:
                    blur(s_next)

        sum_ref[0, 0] = jnp.sum(vsum)

    return body


def _ssim3d(img1, img2):
    N, C, D, H, W = img1.shape
    B = N * C
    g = _gauss_taps()
    tw = _lane_blur_mat(g, W)
    tw2 = np.zeros((2 * W, 2 * W), np.float32)
    tw2[:W, :W] = tw
    tw2[W:, W:] = tw
    ah = np.ascontiguousarray(_lane_blur_mat(g, H).T)

    x = img1.astype(jnp.float32).reshape(B, D, H, W)
    y = img2.astype(jnp.float32).reshape(B, D, H, W)
    xy = jnp.concatenate([x, y], axis=-1)

    body = _make_body(tuple(float(v) for v in g), D, H, W)

    neg_map, psums = pl.pallas_call(
        body,
        grid=(B,),
        in_specs=[
            pl.BlockSpec((None, D, H, 2 * W), lambda b: (b, 0, 0, 0)),
            pl.BlockSpec((2 * W, 2 * W), lambda b: (0, 0)),
            pl.BlockSpec((H, H), lambda b: (0, 0)),
        ],
        out_specs=(
            pl.BlockSpec((None, D, H, W), lambda b: (b, 0, 0, 0)),
            pl.BlockSpec((None, 1, 1), lambda b: (b, 0, 0),
                         memory_space=pltpu.MemorySpace.SMEM),
        ),
        out_shape=(
            jax.ShapeDtypeStruct((B, D, H, W), jnp.float32),
            jax.ShapeDtypeStruct((B, 1, 1), jnp.float32),
        ),
        scratch_shapes=[
            pltpu.VMEM((D, 3 * H, 2 * W), jnp.float32),
        ],
        compiler_params=pltpu.CompilerParams(
            dimension_semantics=("parallel",),
            vmem_limit_bytes=56 * 1024 * 1024,
        ),
    )(jnp.asarray(xy), jnp.asarray(tw2), jnp.asarray(ah))

    mean = jnp.sum(psums) / float(B * D * H * W)
    return 1.0 - mean, neg_map.reshape(N, C, D, H, W)


def kernel(img1, img2):
    return _ssim3d(img1, img2)
